# baseline (speedup 1.0000x reference)
import functools

import jax
import jax.numpy as jnp
from jax import lax
from jax.experimental import pallas as pl
from jax.experimental.pallas import tpu as pltpu
from jax.experimental.pallas import tpu_sc as plsc

NUM_ENTITIES = 1000000
NUM_RELATIONS = 1000
EMBED_DIM = 64
HALF_DIM = EMBED_DIM // 2
BATCH = 16384

NC = 2
NS = 16
NW = NC * NS
B_PER_W = BATCH // NW
CHUNK = 128
N_CHUNKS = B_PER_W // CHUNK
N_GROUPS = B_PER_W // 16


def _rot_body(p_ref, o_ref):
    ph = p_ref[...]
    o_ref[...] = jnp.concatenate([jnp.cos(ph), jnp.sin(ph)], axis=-1)


def _rsqrt(x):
    i = plsc.bitcast(x, jnp.int32)
    i = jnp.int32(0x5F3759DF) - lax.shift_right_logical(i, 1)
    y = plsc.bitcast(i, jnp.float32)
    half = jnp.float32(0.5) * x
    for _ in range(2):
        y = y * (jnp.float32(1.5) - half * y * y)
    return y


def _sc_body(ent_hbm, rot_hbm, heads_hbm, tails_hbm, rels_hbm, out_hbm,
             idx_h, idx_t, idx_r, h_rows, t_rows, r_rows, out_v, sem):
    wid = lax.axis_index("s") * NC + lax.axis_index("c")

    pltpu.sync_copy(heads_hbm.at[wid], idx_h)
    pltpu.sync_copy(tails_hbm.at[wid], idx_t)
    pltpu.sync_copy(rels_hbm.at[wid], idx_r)

    copies = []
    for j in range(N_CHUNKS):
        dst = pl.ds(j * CHUNK, CHUNK)
        copies.append(pltpu.async_copy(ent_hbm.at[idx_h.at[j]], h_rows.at[dst], sem))
        copies.append(pltpu.async_copy(ent_hbm.at[idx_t.at[j]], t_rows.at[dst], sem))
        copies.append(pltpu.async_copy(rot_hbm.at[idx_r.at[j]], r_rows.at[dst], sem))
    for c in copies:
        c.wait()

    eps = jnp.float32(1e-12)

    lane = lax.iota(jnp.int32, 16)
    zeros = jnp.zeros((16,), jnp.float32)

    def group_body(g, _):
        def row_body(k, score):
            i = g * 16 + k
            acc = None
            for off in (0, 16):
                h_re = h_rows[i, pl.ds(off, 16)]
                h_im = h_rows[i, pl.ds(off + HALF_DIM, 16)]
                t_re = t_rows[i, pl.ds(off, 16)]
                t_im = t_rows[i, pl.ds(off + HALF_DIM, 16)]
                c_re = r_rows[i, pl.ds(off, 16)]
                c_im = r_rows[i, pl.ds(off + HALF_DIM, 16)]
                diff_re = h_re * c_re - h_im * c_im - t_re
                diff_im = h_re * c_im + h_im * c_re - t_im
                sq = diff_re * diff_re + diff_im * diff_im + eps
                mag = sq * _rsqrt(sq)
                acc = mag if acc is None else acc + mag
            s = jnp.full((16,), jnp.sum(acc), jnp.float32)
            return jnp.where(lane == k, s, score)

        score = lax.fori_loop(0, 16, row_body, zeros)
        out_v[pl.ds(g * 16, 16)] = -score
        return 0

    lax.fori_loop(0, N_GROUPS, group_body, 0)

    pltpu.sync_copy(out_v, out_hbm.at[pl.ds(wid * B_PER_W, B_PER_W)])


_rot_call = pl.pallas_call(
    _rot_body,
    out_shape=jax.ShapeDtypeStruct((NUM_RELATIONS, EMBED_DIM), jnp.float32),
)

@functools.lru_cache(maxsize=1)
def _sc_call():
    return pl.kernel(
        _sc_body,
        out_type=jax.ShapeDtypeStruct((BATCH,), jnp.float32),
        mesh=plsc.VectorSubcoreMesh(core_axis_name="c", subcore_axis_name="s",
                                    num_cores=NC, num_subcores=NS),
        compiler_params=pltpu.CompilerParams(needs_layout_passes=False,
                                             use_tc_tiling_on_sc=False),
        scratch_types=[
            pltpu.VMEM((N_CHUNKS, CHUNK), jnp.int32),
            pltpu.VMEM((N_CHUNKS, CHUNK), jnp.int32),
            pltpu.VMEM((N_CHUNKS, CHUNK), jnp.int32),
            pltpu.VMEM((B_PER_W, EMBED_DIM), jnp.float32),
            pltpu.VMEM((B_PER_W, EMBED_DIM), jnp.float32),
            pltpu.VMEM((B_PER_W, EMBED_DIM), jnp.float32),
            pltpu.VMEM((B_PER_W,), jnp.float32),
            pltpu.SemaphoreType.DMA,
        ],
    )


@jax.jit
def kernel(entity_emb, heads, relations, tails, relation_phase_weight):
    rot = _rot_call(relation_phase_weight)
    heads3 = heads.astype(jnp.int32).reshape(NW, N_CHUNKS, CHUNK)
    tails3 = tails.astype(jnp.int32).reshape(NW, N_CHUNKS, CHUNK)
    rels3 = relations.astype(jnp.int32).reshape(NW, N_CHUNKS, CHUNK)
    return _sc_call()(entity_emb, rot, heads3, tails3, rels3)


# speedup vs baseline: 1.4416x; 1.4416x over previous
import functools

import jax
import jax.numpy as jnp
from jax import lax
from jax.experimental import pallas as pl
from jax.experimental.pallas import tpu as pltpu
from jax.experimental.pallas import tpu_sc as plsc

NUM_ENTITIES = 1000000
NUM_RELATIONS = 1000
EMBED_DIM = 64
HALF_DIM = EMBED_DIM // 2
BATCH = 16384

NC = 2
NS = 16
NW = NC * NS
B_PER_W = BATCH // NW
PHASE = 32
N_PHASES = B_PER_W // PHASE
WAVE = 16
ROT_ROW = 2 * EMBED_DIM


def _rot_body(p_ref, o_ref):
    ph = p_ref[...]
    z = jnp.zeros_like(ph)
    o_ref[...] = jnp.concatenate([jnp.cos(ph), jnp.sin(ph), z, z], axis=-1)


def _rsqrt(x):
    i = plsc.bitcast(x, jnp.int32)
    i = jnp.int32(0x5F3759DF) - lax.shift_right_logical(i, 1)
    y = plsc.bitcast(i, jnp.float32)
    half = jnp.float32(0.5) * x
    for _ in range(2):
        y = y * (jnp.float32(1.5) - half * y * y)
    return y


def _sc_body(ent_hbm, rot_hbm, heads_hbm, tails_hbm, rels_hbm, out_hbm,
             idx_h, idx_t, idx_r, h8, t8, rr, out_v, sem, rsem):
    wid = lax.axis_index("s") * NC + lax.axis_index("c")
    base = wid * B_PER_W

    pltpu.sync_copy(heads_hbm.at[pl.ds(base, B_PER_W)], idx_h)
    pltpu.sync_copy(tails_hbm.at[pl.ds(base, B_PER_W)], idx_t)
    pltpu.sync_copy(rels_hbm.at[wid], idx_r)

    eps = jnp.float32(1e-12)
    lane = lax.iota(jnp.int32, 16)
    zeros = jnp.zeros((16,), jnp.float32)
    seven = jnp.full((16,), 7, jnp.int32)

    def phase_body(ph, _):
        p0 = ph * PHASE
        rot_cp = pltpu.async_copy(rot_hbm.at[idx_r.at[ph]], rr, rsem)

        def wave_body(w, _):
            i0 = w * WAVE
            hv = idx_h[pl.ds(p0 + i0, WAVE)]
            tv = idx_t[pl.ds(p0 + i0, WAVE)]
            hb = lax.shift_right_logical(hv, 3) * 8
            tb = lax.shift_right_logical(tv, 3) * 8
            cs = []
            for k in range(WAVE):
                cs.append(pltpu.async_copy(
                    ent_hbm.at[pl.ds(pl.multiple_of(hb[k], 8), 8)],
                    h8.at[i0 + k], sem))
                cs.append(pltpu.async_copy(
                    ent_hbm.at[pl.ds(pl.multiple_of(tb[k], 8), 8)],
                    t8.at[i0 + k], sem))
            for c in cs:
                c.wait()
            return 0

        lax.fori_loop(0, PHASE // WAVE, wave_body, 0)
        rot_cp.wait()

        def group_body(g, _):
            i0 = g * 16
            rh = idx_h[pl.ds(p0 + i0, 16)] & seven
            rt = idx_t[pl.ds(p0 + i0, 16)] & seven

            def rows():
                score = zeros
                for k in range(16):
                    i = i0 + k
                    acc = None
                    for off in (0, 16):
                        h_re = h8[i, rh[k], pl.ds(off, 16)]
                        h_im = h8[i, rh[k], pl.ds(HALF_DIM + off, 16)]
                        t_re = t8[i, rt[k], pl.ds(off, 16)]
                        t_im = t8[i, rt[k], pl.ds(HALF_DIM + off, 16)]
                        c_re = rr[i, pl.ds(off, 16)]
                        c_im = rr[i, pl.ds(HALF_DIM + off, 16)]
                        diff_re = h_re * c_re - h_im * c_im - t_re
                        diff_im = h_re * c_im + h_im * c_re - t_im
                        sq = diff_re * diff_re + diff_im * diff_im + eps
                        mag = sq * _rsqrt(sq)
                        acc = mag if acc is None else acc + mag
                    s = jnp.full((16,), jnp.sum(acc), jnp.float32)
                    score = jnp.where(lane == k, s, score)
                return score

            out_v[pl.ds(p0 + i0, 16)] = -rows()
            return 0

        lax.fori_loop(0, PHASE // 16, group_body, 0)
        return 0

    lax.fori_loop(0, N_PHASES, phase_body, 0)

    pltpu.sync_copy(out_v, out_hbm.at[pl.ds(base, B_PER_W)])


@functools.lru_cache(maxsize=1)
def _sc_call():
    return pl.kernel(
        _sc_body,
        out_type=jax.ShapeDtypeStruct((BATCH,), jnp.float32),
        mesh=plsc.VectorSubcoreMesh(core_axis_name="c", subcore_axis_name="s",
                                    num_cores=NC, num_subcores=NS),
        compiler_params=pltpu.CompilerParams(needs_layout_passes=False,
                                             use_tc_tiling_on_sc=True),
        scratch_types=[
            pltpu.VMEM((B_PER_W,), jnp.int32),
            pltpu.VMEM((B_PER_W,), jnp.int32),
            pltpu.VMEM((N_PHASES, PHASE), jnp.int32),
            pltpu.VMEM((PHASE, 8, EMBED_DIM), jnp.float32),
            pltpu.VMEM((PHASE, 8, EMBED_DIM), jnp.float32),
            pltpu.VMEM((PHASE, ROT_ROW), jnp.float32),
            pltpu.VMEM((B_PER_W,), jnp.float32),
            pltpu.SemaphoreType.DMA,
            pltpu.SemaphoreType.DMA,
        ],
    )


_rot_call = pl.pallas_call(
    _rot_body,
    out_shape=jax.ShapeDtypeStruct((NUM_RELATIONS, ROT_ROW), jnp.float32),
)


@jax.jit
def kernel(entity_emb, heads, relations, tails, relation_phase_weight):
    rot = _rot_call(relation_phase_weight)
    rels3 = relations.astype(jnp.int32).reshape(NW, N_PHASES, PHASE)
    return _sc_call()(entity_emb, rot, heads.astype(jnp.int32),
                      tails.astype(jnp.int32), rels3)


# speedup vs baseline: 2.0370x; 1.4130x over previous
import functools

import jax
import jax.numpy as jnp
from jax import lax
from jax.experimental import pallas as pl
from jax.experimental.pallas import tpu as pltpu
from jax.experimental.pallas import tpu_sc as plsc
from jax.experimental.layout import Layout, with_layout_constraint

NUM_ENTITIES = 1000000
NUM_RELATIONS = 1000
EMBED_DIM = 64
HALF_DIM = EMBED_DIM // 2
BATCH = 16384

NC = 2
NS = 16
NW = NC * NS
B_PER_W = BATCH // NW
PHASE = 32
N_PHASES = B_PER_W // PHASE
WAVE = 16
ROT_ROW = 2 * EMBED_DIM


def _rot_body(p_ref, o_ref):
    ph = p_ref[...]
    z = jnp.zeros_like(ph)
    o_ref[...] = jnp.concatenate([jnp.cos(ph), jnp.sin(ph), z, z], axis=-1)


def _rsqrt(x):
    i = plsc.bitcast(x, jnp.int32)
    i = jnp.int32(0x5F3759DF) - lax.shift_right_logical(i, 1)
    y = plsc.bitcast(i, jnp.float32)
    half = jnp.float32(0.5) * x
    for _ in range(2):
        y = y * (jnp.float32(1.5) - half * y * y)
    return y


def _sc_body(ent_hbm, rot_hbm, heads_hbm, tails_hbm, rels_hbm, out_hbm,
             idx_h, idx_t, idx_r, h8, t8, rr, out_v, sem, rsem):
    wid = lax.axis_index("s") * NC + lax.axis_index("c")
    base = wid * B_PER_W

    pltpu.sync_copy(heads_hbm.at[pl.ds(base, B_PER_W)], idx_h)
    pltpu.sync_copy(tails_hbm.at[pl.ds(base, B_PER_W)], idx_t)
    pltpu.sync_copy(rels_hbm.at[wid], idx_r)

    eps = jnp.float32(1e-12)
    lane = lax.iota(jnp.int32, 16)
    zeros = jnp.zeros((16,), jnp.float32)
    seven = jnp.full((16,), 7, jnp.int32)

    def phase_body(ph, _):
        p0 = ph * PHASE
        rot_cp = pltpu.async_copy(rot_hbm.at[idx_r.at[ph]], rr, rsem)

        def wave_body(w, _):
            i0 = w * WAVE
            hv = idx_h[pl.ds(p0 + i0, WAVE)]
            tv = idx_t[pl.ds(p0 + i0, WAVE)]
            hb = lax.shift_right_logical(hv, 3)
            tb = lax.shift_right_logical(tv, 3)
            cs = []
            for k in range(WAVE):
                cs.append(pltpu.async_copy(
                    ent_hbm.at[hb[k]], h8.at[i0 + k], sem))
                cs.append(pltpu.async_copy(
                    ent_hbm.at[tb[k]], t8.at[i0 + k], sem))
            for c in cs:
                c.wait()
            return 0

        lax.fori_loop(0, PHASE // WAVE, wave_body, 0)
        rot_cp.wait()

        def group_body(g, _):
            i0 = g * 16
            rh = idx_h[pl.ds(p0 + i0, 16)] & seven
            rt = idx_t[pl.ds(p0 + i0, 16)] & seven

            def rows():
                score = zeros
                for k in range(16):
                    i = i0 + k
                    acc = None
                    for off in (0, 16):
                        h_re = h8[i, rh[k], pl.ds(off, 16)]
                        h_im = h8[i, rh[k], pl.ds(HALF_DIM + off, 16)]
                        t_re = t8[i, rt[k], pl.ds(off, 16)]
                        t_im = t8[i, rt[k], pl.ds(HALF_DIM + off, 16)]
                        c_re = rr[i, pl.ds(off, 16)]
                        c_im = rr[i, pl.ds(HALF_DIM + off, 16)]
                        diff_re = h_re * c_re - h_im * c_im - t_re
                        diff_im = h_re * c_im + h_im * c_re - t_im
                        sq = diff_re * diff_re + diff_im * diff_im + eps
                        mag = sq * _rsqrt(sq)
                        acc = mag if acc is None else acc + mag
                    s = jnp.full((16,), jnp.sum(acc), jnp.float32)
                    score = jnp.where(lane == k, s, score)
                return score

            out_v[pl.ds(p0 + i0, 16)] = -rows()
            return 0

        lax.fori_loop(0, PHASE // 16, group_body, 0)
        return 0

    lax.fori_loop(0, N_PHASES, phase_body, 0)

    pltpu.sync_copy(out_v, out_hbm.at[pl.ds(base, B_PER_W)])


@functools.lru_cache(maxsize=1)
def _sc_call():
    return pl.kernel(
        _sc_body,
        out_type=jax.ShapeDtypeStruct((BATCH,), jnp.float32),
        mesh=plsc.VectorSubcoreMesh(core_axis_name="c", subcore_axis_name="s",
                                    num_cores=NC, num_subcores=NS),
        compiler_params=pltpu.CompilerParams(needs_layout_passes=False,
                                             use_tc_tiling_on_sc=True),
        scratch_types=[
            pltpu.VMEM((B_PER_W,), jnp.int32),
            pltpu.VMEM((B_PER_W,), jnp.int32),
            pltpu.VMEM((N_PHASES, PHASE), jnp.int32),
            pltpu.VMEM((PHASE, 8, EMBED_DIM), jnp.float32),
            pltpu.VMEM((PHASE, 8, EMBED_DIM), jnp.float32),
            pltpu.VMEM((PHASE, ROT_ROW), jnp.float32),
            pltpu.VMEM((B_PER_W,), jnp.float32),
            pltpu.SemaphoreType.DMA,
            pltpu.SemaphoreType.DMA,
        ],
    )


_rot_call = pl.pallas_call(
    _rot_body,
    out_shape=jax.ShapeDtypeStruct((NUM_RELATIONS, ROT_ROW), jnp.float32),
)


@jax.jit
def kernel(entity_emb, heads, relations, tails, relation_phase_weight):
    rot = _rot_call(relation_phase_weight)
    rels3 = relations.astype(jnp.int32).reshape(NW, N_PHASES, PHASE)
    ent3 = entity_emb.reshape(NUM_ENTITIES // 8, 8, EMBED_DIM)
    return _sc_call()(ent3, rot, heads.astype(jnp.int32),
                      tails.astype(jnp.int32), rels3)


# speedup vs baseline: 2.1287x; 1.0450x over previous
import functools

import jax
import jax.numpy as jnp
from jax import lax
from jax.experimental import pallas as pl
from jax.experimental.pallas import tpu as pltpu
from jax.experimental.pallas import tpu_sc as plsc

NUM_ENTITIES = 1000000
NUM_RELATIONS = 1000
EMBED_DIM = 64
HALF_DIM = EMBED_DIM // 2
BATCH = 16384

NC = 2
NS = 16
NW = NC * NS
B_PER_W = BATCH // NW
PHASE = 16
N_PHASES = B_PER_W // PHASE
ROT_ROW = 2 * EMBED_DIM


def _rot_body(p_ref, o_ref):
    ph = p_ref[...]
    z = jnp.zeros_like(ph)
    o_ref[...] = jnp.concatenate([jnp.cos(ph), jnp.sin(ph), z, z], axis=-1)


def _rsqrt(x):
    i = plsc.bitcast(x, jnp.int32)
    i = jnp.int32(0x5F3759DF) - lax.shift_right_logical(i, 1)
    y = plsc.bitcast(i, jnp.float32)
    half = jnp.float32(0.5) * x
    for _ in range(2):
        y = y * (jnp.float32(1.5) - half * y * y)
    return y


def _sc_body(ent_hbm, rot_hbm, heads_hbm, tails_hbm, rels_hbm, out_hbm,
             idx_h, idx_t, idx_r, h8, t8, rr, out_v, sem_a, sem_b,
             rsem_a, rsem_b):
    wid = lax.axis_index("s") * NC + lax.axis_index("c")
    base = wid * B_PER_W

    pltpu.sync_copy(heads_hbm.at[pl.ds(base, B_PER_W)], idx_h)
    pltpu.sync_copy(tails_hbm.at[pl.ds(base, B_PER_W)], idx_t)
    pltpu.sync_copy(rels_hbm.at[wid], idx_r)

    eps = jnp.float32(1e-12)
    lane = lax.iota(jnp.int32, 16)
    zeros = jnp.zeros((16,), jnp.float32)
    seven = jnp.full((16,), 7, jnp.int32)

    def fire(ph, slot, sem, rsem):
        s0 = slot * PHASE
        pltpu.async_copy(rot_hbm.at[idx_r.at[ph]],
                         rr.at[pl.ds(s0, PHASE)], rsem)
        hv = idx_h[pl.ds(ph * PHASE, PHASE)]
        tv = idx_t[pl.ds(ph * PHASE, PHASE)]
        hb = lax.shift_right_logical(hv, 3)
        tb = lax.shift_right_logical(tv, 3)
        for k in range(PHASE):
            pltpu.async_copy(ent_hbm.at[hb[k]], h8.at[s0 + k], sem)
            pltpu.async_copy(ent_hbm.at[tb[k]], t8.at[s0 + k], sem)

    def drain(sem, rsem):
        pltpu.make_async_copy(rot_hbm.at[pl.ds(0, PHASE)],
                              rr.at[pl.ds(0, PHASE)], rsem).wait()
        for k in range(PHASE):
            pltpu.make_async_copy(ent_hbm.at[0], h8.at[k], sem).wait()
            pltpu.make_async_copy(ent_hbm.at[0], t8.at[k], sem).wait()

    def compute(ph, slot):
        s0 = slot * PHASE
        rh = idx_h[pl.ds(ph * PHASE, 16)] & seven
        rt = idx_t[pl.ds(ph * PHASE, 16)] & seven
        score = zeros
        for k in range(16):
            i = s0 + k
            acc = None
            for off in (0, 16):
                h_re = h8[i, rh[k], pl.ds(off, 16)]
                h_im = h8[i, rh[k], pl.ds(HALF_DIM + off, 16)]
                t_re = t8[i, rt[k], pl.ds(off, 16)]
                t_im = t8[i, rt[k], pl.ds(HALF_DIM + off, 16)]
                c_re = rr[i, pl.ds(off, 16)]
                c_im = rr[i, pl.ds(HALF_DIM + off, 16)]
                diff_re = h_re * c_re - h_im * c_im - t_re
                diff_im = h_re * c_im + h_im * c_re - t_im
                sq = diff_re * diff_re + diff_im * diff_im + eps
                mag = sq * _rsqrt(sq)
                acc = mag if acc is None else acc + mag
            s = jnp.full((16,), jnp.sum(acc), jnp.float32)
            score = jnp.where(lane == k, s, score)
        out_v[pl.ds(ph * PHASE, 16)] = -score

    fire(0, 0, sem_a, rsem_a)

    def phase_body(ph, _):
        even = ph % 2 == 0

        @pl.when(ph + 1 < N_PHASES)
        def _():
            @pl.when(even)
            def _():
                fire(ph + 1, 1, sem_b, rsem_b)

            @pl.when(jnp.logical_not(even))
            def _():
                fire(ph + 1, 0, sem_a, rsem_a)

        @pl.when(even)
        def _():
            drain(sem_a, rsem_a)
            compute(ph, 0)

        @pl.when(jnp.logical_not(even))
        def _():
            drain(sem_b, rsem_b)
            compute(ph, 1)

        return 0

    lax.fori_loop(0, N_PHASES, phase_body, 0)

    pltpu.sync_copy(out_v, out_hbm.at[pl.ds(base, B_PER_W)])


@functools.lru_cache(maxsize=1)
def _sc_call():
    return pl.kernel(
        _sc_body,
        out_type=jax.ShapeDtypeStruct((BATCH,), jnp.float32),
        mesh=plsc.VectorSubcoreMesh(core_axis_name="c", subcore_axis_name="s",
                                    num_cores=NC, num_subcores=NS),
        compiler_params=pltpu.CompilerParams(needs_layout_passes=False,
                                             use_tc_tiling_on_sc=True),
        scratch_types=[
            pltpu.VMEM((B_PER_W,), jnp.int32),
            pltpu.VMEM((B_PER_W,), jnp.int32),
            pltpu.VMEM((N_PHASES, PHASE), jnp.int32),
            pltpu.VMEM((2 * PHASE, 8, EMBED_DIM), jnp.float32),
            pltpu.VMEM((2 * PHASE, 8, EMBED_DIM), jnp.float32),
            pltpu.VMEM((2 * PHASE, ROT_ROW), jnp.float32),
            pltpu.VMEM((B_PER_W,), jnp.float32),
            pltpu.SemaphoreType.DMA,
            pltpu.SemaphoreType.DMA,
            pltpu.SemaphoreType.DMA,
            pltpu.SemaphoreType.DMA,
        ],
    )


_rot_call = pl.pallas_call(
    _rot_body,
    out_shape=jax.ShapeDtypeStruct((NUM_RELATIONS, ROT_ROW), jnp.float32),
)


@jax.jit
def kernel(entity_emb, heads, relations, tails, relation_phase_weight):
    rot = _rot_call(relation_phase_weight)
    rels3 = relations.astype(jnp.int32).reshape(NW, N_PHASES, PHASE)
    ent3 = entity_emb.reshape(NUM_ENTITIES // 8, 8, EMBED_DIM)
    return _sc_call()(ent3, rot, heads.astype(jnp.int32),
                      tails.astype(jnp.int32), rels3)
